# baseline (speedup 1.0000x reference)
import functools

import jax
import jax.numpy as jnp
from jax import lax
from jax.experimental import pallas as pl
from jax.experimental.pallas import tpu as pltpu
from jax.experimental.pallas import tpu_sc as plsc

N = 10000
E = 320000
D = 128
F = D // 2

NC = 2
NS = 16
K = 128
NCH = E // K
CPT = NCH // NS
XT2 = NCH - NS * CPT
DB = 168
NCHP = 2512
NB = 2
NACC = 10240
RPT = NACC // NS


@functools.cache
def _make_sc_segsum(with_deg):
    mesh = plsc.VectorSubcoreMesh(
        core_axis_name="c", subcore_axis_name="s",
        num_cores=NC, num_subcores=NS,
    )
    out_type = [jax.ShapeDtypeStruct((NC, NACC, F), jnp.float32)]
    scratch = (
        [pltpu.VMEM((K,), jnp.int32) for _ in range(NB)]
        + [pltpu.VMEM((K, F), jnp.float32) for _ in range(NB)]
        + [
            pltpu.VMEM((DB, K), jnp.int32),
            pltpu.VMEM((16, F), jnp.float32),
            pltpu.VMEM_SHARED((NACC, F), jnp.float32),
            pltpu.VMEM_SHARED((NACC, F), jnp.float32),
        ]
        + [pltpu.SemaphoreType.DMA for _ in range(NB)]
        + [pltpu.SemaphoreType.DMA for _ in range(NB)]
        + [
            pltpu.SemaphoreType.DMA,
            pltpu.SemaphoreType.DMA,
        ]
    )
    if with_deg:
        out_type.append(jax.ShapeDtypeStruct((NC, NACC), jnp.float32))
        scratch += (
            [
                pltpu.VMEM((K,), jnp.float32),
                pltpu.VMEM((RPT,), jnp.float32),
                pltpu.VMEM_SHARED((NACC,), jnp.float32),
            ]
            + [pltpu.SemaphoreType.DMA for _ in range(NB)]
        )
    return pl.kernel(
        functools.partial(_sc_segsum_body, with_deg),
        out_type=out_type,
        mesh=mesh,
        scratch_types=scratch,
        compiler_params=pltpu.CompilerParams(use_tc_tiling_on_sc=False),
    )


def _sc_segsum_body(with_deg, ga, gb, srcp, dst2d, out, *rest):
    if with_deg:
        dout = rest[0]
        rest = rest[1:]
    sidx = rest[0:NB]
    rows = rest[NB:2 * NB]
    dbuf, zbuf, acc, stable = rest[2 * NB:2 * NB + 4]
    sg = rest[2 * NB + 4:3 * NB + 4]
    ss = rest[3 * NB + 4:4 * NB + 4]
    si, st = rest[4 * NB + 4:4 * NB + 6]
    if with_deg:
        ones, dzero, dacc = rest[4 * NB + 6:4 * NB + 9]
        sd = rest[4 * NB + 9:5 * NB + 9]

    c = lax.axis_index("c")
    s = lax.axis_index("s")

    cs = s * CPT + jnp.minimum(s, XT2)
    cs8 = (cs // 8) * 8
    off = cs - cs8
    e0 = cs * K
    has_x = s < XT2

    def load_src(t, b):
        pltpu.async_copy(srcp.at[pl.ds(e0 + t * K, K)], sidx[b], si)

    def wait_src(b):
        pltpu.make_async_copy(srcp.at[pl.ds(e0, K)], sidx[b], si).wait()

    pltpu.async_copy(dst2d.at[pl.ds(cs8, DB)], dbuf, st)
    load_src(0, 0)
    srow0 = s * RPT

    @pl.when(c == 0)
    def _stage_a_half():
        pltpu.async_copy(ga.at[pl.ds(srow0, RPT)],
                         stable.at[pl.ds(srow0, RPT)], st)

    @pl.when(c == 1)
    def _stage_b_half():
        pltpu.async_copy(gb.at[pl.ds(srow0, RPT)],
                         stable.at[pl.ds(srow0, RPT)], st)

    z16 = jnp.zeros((16,), jnp.float32)
    for i in range(16):
        for j in range(F // 16):
            zbuf[i, pl.ds(j * 16, 16)] = z16
    row0 = s * RPT

    @pl.loop(0, RPT // 16)
    def _zero(j):
        pltpu.sync_copy(zbuf, acc.at[pl.ds(row0 + j * 16, 16)])

    if with_deg:
        o16 = jnp.ones((16,), jnp.float32)
        for j in range(K // 16):
            ones[pl.ds(j * 16, 16)] = o16
        for j in range(RPT // 16):
            dzero[pl.ds(j * 16, 16)] = z16
        pltpu.sync_copy(dzero, dacc.at[pl.ds(row0, RPT)])

    plsc.subcore_barrier()

    pltpu.make_async_copy(dst2d.at[pl.ds(cs8, DB)], dbuf, st).wait()
    pltpu.make_async_copy(ga.at[pl.ds(srow0, RPT)],
                          stable.at[pl.ds(srow0, RPT)], st).wait()
    wait_src(0)

    def start_gather(b):
        pltpu.async_copy(stable.at[sidx[b]], rows[b], sg[b])

    def wait_gather(b):
        pltpu.make_async_copy(stable.at[sidx[b]], rows[b], sg[b]).wait()

    def start_scatter(t, b):
        pltpu.async_copy(rows[b], acc.at[dbuf.at[off + t]], ss[b], add=True)

    def wait_scatter(b):
        pltpu.make_async_copy(rows[b], acc.at[dbuf.at[0]], ss[b]).wait()

    if with_deg:
        def start_deg(t, b):
            pltpu.async_copy(ones, dacc.at[dbuf.at[off + t]], sd[b], add=True)

        def wait_deg(b):
            pltpu.make_async_copy(ones, dacc.at[dbuf.at[0]], sd[b]).wait()

    start_gather(0)
    for j in range(1, NB - 1):
        load_src(j, j)
        wait_src(j)
        start_gather(j)

    def step(t, b, first, cond_prep):
        wait_gather(b)
        start_scatter(t, b)
        if with_deg:
            if not first:
                wait_deg(b)
            start_deg(t, b)

        def _p():
            bj = (b + NB - 1) % NB
            load_src(t + NB - 1, bj)
            if not (first and b == 0):
                wait_scatter(bj)
            wait_src(bj)
            start_gather(bj)

        if cond_prep:
            pl.when(t + NB - 1 < CPT)(_p)
        else:
            _p()

    for t in range(NB):
        step(t, t, True, False)

    @pl.loop(NB, CPT, step=NB)
    def _body(t0):
        for b in range(NB):
            step(t0 + b, b, False, True)

    for b in range(NB):
        wait_scatter(b)
        if with_deg:
            wait_deg(b)

    @pl.when(has_x)
    def _extra():
        pltpu.async_copy(srcp.at[pl.ds(e0 + CPT * K, K)], sidx[0], st).wait()
        pltpu.async_copy(stable.at[sidx[0]], rows[0], st).wait()
        pltpu.sync_copy(rows[0], acc.at[dbuf.at[off + CPT]], add=True)
        if with_deg:
            pltpu.async_copy(ones, dacc.at[dbuf.at[off + CPT]], st,
                             add=True).wait()

    plsc.subcore_barrier()
    pltpu.sync_copy(acc.at[pl.ds(row0, RPT)], out.at[c, pl.ds(row0, RPT)])
    if with_deg:
        pltpu.sync_copy(dacc.at[pl.ds(row0, RPT)],
                        dout.at[c, pl.ds(row0, RPT)])


def _stage_a_body(x_ref, wl_ref, wr_ref, ga_ref, gb_ref, r_ref):
    x = x_ref[...]
    g = jnp.dot(x, wl_ref[...], preferred_element_type=jnp.float32)
    ga_ref[pl.ds(0, N), :] = g[:, :F]
    gb_ref[pl.ds(0, N), :] = g[:, F:]
    r_ref[...] = jnp.dot(x, wr_ref[...], preferred_element_type=jnp.float32)


_stage_a = pl.pallas_call(
    _stage_a_body,
    out_shape=[
        jax.ShapeDtypeStruct((NACC, F), jnp.float32),
        jax.ShapeDtypeStruct((NACC, F), jnp.float32),
        jax.ShapeDtypeStruct((N, D), jnp.float32),
    ],
)


def _hidden(acc_ref, deg_ref, r_ref, b_ref):
    ssum = jnp.concatenate([acc_ref[0, :N, :], acc_ref[1, :N, :]], axis=1)
    deg = jnp.maximum(deg_ref[:N, :], 1.0)
    return jnp.maximum(ssum / deg + b_ref[...] + r_ref[...], 0.0)


def _stage_c_body(acc_ref, deg_ref, r_ref, b_ref, wl_ref, wr_ref,
                  ga_ref, gb_ref, r2_ref):
    h = _hidden(acc_ref, deg_ref, r_ref, b_ref)
    g2 = jnp.dot(h, wl_ref[...], preferred_element_type=jnp.float32)
    ga_ref[pl.ds(0, N), :] = g2[:, :F]
    gb_ref[pl.ds(0, N), :] = g2[:, F:]
    r2_ref[...] = jnp.dot(h, wr_ref[...], preferred_element_type=jnp.float32)


_stage_c = pl.pallas_call(
    _stage_c_body,
    out_shape=[
        jax.ShapeDtypeStruct((NACC, F), jnp.float32),
        jax.ShapeDtypeStruct((NACC, F), jnp.float32),
        jax.ShapeDtypeStruct((N, D), jnp.float32),
    ],
)


def _stage_e_body(acc_ref, deg_ref, r_ref, b_ref, wa_ref, ba_ref,
                  wc_ref, bc_ref, logits_ref, values_ref):
    h = _hidden(acc_ref, deg_ref, r_ref, b_ref)
    logits_ref[...] = (
        jnp.dot(h, wa_ref[...], preferred_element_type=jnp.float32)
        + ba_ref[...]
    )
    values_ref[...] = (
        jnp.dot(h, wc_ref[...], preferred_element_type=jnp.float32)
        + bc_ref[...]
    )


_stage_e = pl.pallas_call(
    _stage_e_body,
    out_shape=[
        jax.ShapeDtypeStruct((N, 64), jnp.float32),
        jax.ShapeDtypeStruct((N, 1), jnp.float32),
    ],
)


def kernel(x, edge_index, W1l, b1, W1r, W2l, b2, W2r, Wa, ba, Wc, bc):
    srcp = edge_index[0].astype(jnp.int32)
    dstp = edge_index[1].astype(jnp.int32)
    dst2d = jnp.concatenate(
        [dstp, jnp.zeros((NCHP * K - E,), jnp.int32)]).reshape(NCHP, K)

    sc_segsum_deg = _make_sc_segsum(True)
    sc_segsum = _make_sc_segsum(False)

    ga1, gb1, r1 = _stage_a(x, W1l, W1r)
    acc1, degs = sc_segsum_deg(ga1, gb1, srcp, dst2d)
    deg = degs[0].reshape(NACC, 1)
    ga2, gb2, r2 = _stage_c(acc1, deg, r1, b1.reshape(1, D), W2l, W2r)
    (acc2,) = sc_segsum(ga2, gb2, srcp, dst2d)
    logits, values = _stage_e(
        acc2, deg, r2, b2.reshape(1, D),
        Wa, ba.reshape(1, 64), Wc, bc.reshape(1, 1),
    )
    return logits, values.reshape(N)


# speedup vs baseline: 1.5387x; 1.5387x over previous
import functools

import jax
import jax.numpy as jnp
from jax import lax
from jax.experimental import pallas as pl
from jax.experimental.pallas import tpu as pltpu
from jax.experimental.pallas import tpu_sc as plsc

N = 10000
E = 320000
D = 128

NC = 2
NS = 16
NW = NC * NS
K = 128
NCH = E // K
CPW = NCH // NW
XTRA = NCH - NW * CPW
DB = CPW + 10
NCHP = NCH + 8
NACC = 10240
RPT = NACC // NS


@functools.cache
def _make_sc_segsum(with_deg):
    mesh = plsc.VectorSubcoreMesh(
        core_axis_name="c", subcore_axis_name="s",
        num_cores=NC, num_subcores=NS,
    )
    out_type = [jax.ShapeDtypeStruct((NC, NACC, D), jnp.float32)]
    scratch = [
        pltpu.VMEM((K,), jnp.int32),
        pltpu.VMEM((K,), jnp.int32),
        pltpu.VMEM((DB, K), jnp.int32),
        pltpu.VMEM((K, D), jnp.float32),
        pltpu.VMEM((K, D), jnp.float32),
        pltpu.VMEM((16, D), jnp.float32),
        pltpu.VMEM_SHARED((NACC, D), jnp.float32),
        pltpu.SemaphoreType.DMA,
        pltpu.SemaphoreType.DMA,
        pltpu.SemaphoreType.DMA,
        pltpu.SemaphoreType.DMA,
        pltpu.SemaphoreType.DMA,
        pltpu.SemaphoreType.DMA,
    ]
    if with_deg:
        out_type.append(jax.ShapeDtypeStruct((NC, NACC), jnp.float32))
        scratch += [
            pltpu.VMEM((K,), jnp.float32),
            pltpu.VMEM((RPT,), jnp.float32),
            pltpu.VMEM_SHARED((NACC,), jnp.float32),
            pltpu.SemaphoreType.DMA,
            pltpu.SemaphoreType.DMA,
        ]
    return pl.kernel(
        functools.partial(_sc_segsum_body, with_deg),
        out_type=out_type,
        mesh=mesh,
        scratch_types=scratch,
    )


def _sc_segsum_body(with_deg, table, srcp, dst2d, out, *rest):
    if with_deg:
        (dout, sidx0, sidx1, dbuf, rows0, rows1, zbuf, acc,
         sg0, sg1, ss0, ss1, si, st, ones, dzero, dacc, sd0, sd1) = rest
        sd = (sd0, sd1)
    else:
        (sidx0, sidx1, dbuf, rows0, rows1, zbuf, acc,
         sg0, sg1, ss0, ss1, si, st) = rest
    c = lax.axis_index("c")
    s = lax.axis_index("s")
    sidx = (sidx0, sidx1)
    rows = (rows0, rows1)
    sg = (sg0, sg1)
    ss = (ss0, ss1)

    w = c * NS + s
    cs = w * CPW + jnp.minimum(w, XTRA)
    cs8 = (cs // 8) * 8
    off = cs - cs8
    e0 = cs * K
    has_x = w < XTRA

    def load_src(t, b):
        pltpu.async_copy(srcp.at[pl.ds(e0 + t * K, K)], sidx[b], si)

    def wait_src(b):
        pltpu.make_async_copy(srcp.at[pl.ds(e0, K)], sidx[b], si).wait()

    pltpu.async_copy(dst2d.at[pl.ds(cs8, DB)], dbuf, st)
    load_src(0, 0)

    z16 = jnp.zeros((16,), jnp.float32)
    for i in range(16):
        for j in range(D // 16):
            zbuf[i, pl.ds(j * 16, 16)] = z16
    row0 = s * RPT

    @pl.loop(0, RPT // 16)
    def _zero(j):
        pltpu.sync_copy(zbuf, acc.at[pl.ds(row0 + j * 16, 16)])

    if with_deg:
        o16 = jnp.ones((16,), jnp.float32)
        for j in range(K // 16):
            ones[pl.ds(j * 16, 16)] = o16
        for j in range(RPT // 16):
            dzero[pl.ds(j * 16, 16)] = z16
        pltpu.sync_copy(dzero, dacc.at[pl.ds(row0, RPT)])

    plsc.subcore_barrier()

    pltpu.make_async_copy(dst2d.at[pl.ds(cs8, DB)], dbuf, st).wait()
    wait_src(0)

    def start_gather(b):
        pltpu.async_copy(table.at[sidx[b]], rows[b], sg[b])

    def wait_gather(b):
        pltpu.make_async_copy(table.at[sidx[b]], rows[b], sg[b]).wait()

    def start_scatter(t, b):
        pltpu.async_copy(rows[b], acc.at[dbuf.at[off + t]], ss[b], add=True)

    def wait_scatter(b):
        pltpu.make_async_copy(rows[b], acc.at[dbuf.at[0]], ss[b]).wait()

    def start_deg(t, b):
        pltpu.async_copy(ones, dacc.at[dbuf.at[off + t]], sd[b], add=True)

    def wait_deg(b):
        pltpu.make_async_copy(ones, dacc.at[dbuf.at[0]], sd[b]).wait()

    start_gather(0)
    load_src(1, 1)
    wait_src(1)
    start_gather(1)

    def step(t, b, first, cond_prep):
        wait_gather(b)
        start_scatter(t, b)
        if with_deg:
            if not first:
                wait_deg(b)
            start_deg(t, b)

        def _p():
            load_src(t + 2, b)
            wait_scatter(b)
            wait_src(b)
            start_gather(b)

        if cond_prep:
            pl.when(t + 2 < CPW)(_p)
        else:
            _p()

    step(0, 0, True, False)
    step(1, 1, True, False)

    @pl.loop(2, CPW, step=2)
    def _body(t0):
        for b in range(2):
            step(t0 + b, b, False, True)

    wait_scatter(0)
    wait_scatter(1)
    if with_deg:
        wait_deg(0)
        wait_deg(1)

    @pl.when(has_x)
    def _extra():
        pltpu.async_copy(srcp.at[pl.ds(e0 + CPW * K, K)], sidx0, st).wait()
        pltpu.async_copy(table.at[sidx0], rows0, st).wait()
        pltpu.sync_copy(rows0, acc.at[dbuf.at[off + CPW]], add=True)
        if with_deg:
            pltpu.async_copy(ones, dacc.at[dbuf.at[off + CPW]], st,
                             add=True).wait()

    plsc.subcore_barrier()
    pltpu.sync_copy(acc.at[pl.ds(row0, RPT)], out.at[c, pl.ds(row0, RPT)])
    if with_deg:
        pltpu.sync_copy(dacc.at[pl.ds(row0, RPT)],
                        dout.at[c, pl.ds(row0, RPT)])


def _stage_a_body(x_ref, wl_ref, wr_ref, g_ref, r_ref):
    x = x_ref[...]
    g_ref[...] = jnp.dot(x, wl_ref[...], preferred_element_type=jnp.float32)
    r_ref[...] = jnp.dot(x, wr_ref[...], preferred_element_type=jnp.float32)


_stage_a = pl.pallas_call(
    _stage_a_body,
    out_shape=[
        jax.ShapeDtypeStruct((N, D), jnp.float32),
        jax.ShapeDtypeStruct((N, D), jnp.float32),
    ],
)


def _stage_c_body(acc_ref, dega_ref, degb_ref, r_ref, b_ref, wl_ref, wr_ref,
                  g2_ref, r2_ref):
    ssum = acc_ref[0, :N, :] + acc_ref[1, :N, :]
    deg = jnp.maximum(dega_ref[:N, :] + degb_ref[:N, :], 1.0)
    h = jnp.maximum(ssum / deg + b_ref[...] + r_ref[...], 0.0)
    g2_ref[...] = jnp.dot(h, wl_ref[...], preferred_element_type=jnp.float32)
    r2_ref[...] = jnp.dot(h, wr_ref[...], preferred_element_type=jnp.float32)


_stage_c = pl.pallas_call(
    _stage_c_body,
    out_shape=[
        jax.ShapeDtypeStruct((N, D), jnp.float32),
        jax.ShapeDtypeStruct((N, D), jnp.float32),
    ],
)


def _stage_e_body(acc_ref, dega_ref, degb_ref, r_ref, b_ref, wa_ref, ba_ref,
                  wc_ref, bc_ref, logits_ref, values_ref):
    ssum = acc_ref[0, :N, :] + acc_ref[1, :N, :]
    deg = jnp.maximum(dega_ref[:N, :] + degb_ref[:N, :], 1.0)
    h = jnp.maximum(ssum / deg + b_ref[...] + r_ref[...], 0.0)
    logits_ref[...] = (
        jnp.dot(h, wa_ref[...], preferred_element_type=jnp.float32)
        + ba_ref[...]
    )
    values_ref[...] = (
        jnp.dot(h, wc_ref[...], preferred_element_type=jnp.float32)
        + bc_ref[...]
    )


_stage_e = pl.pallas_call(
    _stage_e_body,
    out_shape=[
        jax.ShapeDtypeStruct((N, 64), jnp.float32),
        jax.ShapeDtypeStruct((N, 1), jnp.float32),
    ],
)


def kernel(x, edge_index, W1l, b1, W1r, W2l, b2, W2r, Wa, ba, Wc, bc):
    srcp = edge_index[0].astype(jnp.int32)
    dstp = edge_index[1].astype(jnp.int32)
    dst2d = jnp.concatenate(
        [dstp, jnp.zeros((NCHP * K - E,), jnp.int32)]).reshape(NCHP, K)

    sc_segsum_deg = _make_sc_segsum(True)
    sc_segsum = _make_sc_segsum(False)

    g1, r1 = _stage_a(x, W1l, W1r)
    acc1, degs = sc_segsum_deg(g1, srcp, dst2d)
    dega = degs[0].reshape(NACC, 1)
    degb = degs[1].reshape(NACC, 1)
    g2, r2 = _stage_c(acc1, dega, degb, r1, b1.reshape(1, D), W2l, W2r)
    (acc2,) = sc_segsum(g2, srcp, dst2d)
    logits, values = _stage_e(
        acc2, dega, degb, r2, b2.reshape(1, D),
        Wa, ba.reshape(1, 64), Wc, bc.reshape(1, 1),
    )
    return logits, values.reshape(N)
